# baseline (speedup 1.0000x reference)
import functools

import jax
import jax.numpy as jnp
from jax import lax
from jax.experimental import pallas as pl
from jax.experimental.pallas import tpu as pltpu
from jax.experimental.pallas import tpu_sc as plsc

NC = 2
NS = 16
NW = NC * NS
CHUNK = 128
K = 3


def _segment_sum_sc(table, src, dst):
    n, d = table.shape
    e = src.shape[0]
    assert e % CHUNK == 0
    n_chunks = e // CHUNK
    n_iters, rem = divmod(n_chunks, K * NW)
    tail_iters = -(-rem // NW)
    piece = 128
    n_full, tail = divmod(n, piece)
    assert tail % 8 == 0

    mesh = plsc.VectorSubcoreMesh(
        core_axis_name="c", subcore_axis_name="s", num_cores=NC, num_subcores=NS
    )

    @functools.partial(
        pl.kernel,
        out_type=jax.ShapeDtypeStruct((NC, n, d), jnp.float32),
        mesh=mesh,
        scratch_types=[
            [pltpu.VMEM((CHUNK,), jnp.int32) for _ in range(K)],
            [pltpu.VMEM((CHUNK,), jnp.int32) for _ in range(K)],
            [pltpu.VMEM((CHUNK, d), jnp.float32) for _ in range(K)],
            pltpu.VMEM_SHARED((n, d), jnp.float32),
            [pltpu.SemaphoreType.DMA for _ in range(K)],
            [pltpu.SemaphoreType.DMA for _ in range(K)],
            [pltpu.SemaphoreType.DMA for _ in range(K)],
        ],
    )
    def segsum(table_hbm, src_hbm, dst_hbm, out_hbm,
               src_v, dst_v, rows_v, acc_sh, semi, semg, sems):
        c = lax.axis_index("c")
        s = lax.axis_index("s")
        w = c * NS + s

        def zbody(i, carry):
            r = i // (d // 16)
            col = (i % (d // 16)) * 16
            rows_v[0][r, pl.ds(col, 16)] = jnp.zeros((16,), jnp.float32)
            return carry

        lax.fori_loop(0, piece * (d // 16), zbody, 0)

        my_pieces = (n_full - 1 - s) // NS + 1

        def zcopy(i, carry):
            r0 = (s + i * NS) * piece
            pltpu.async_copy(rows_v[0].at[pl.ds(0, piece)],
                             acc_sh.at[pl.ds(r0, piece)], semi[0])
            return carry

        lax.fori_loop(0, my_pieces, zcopy, 0)

        def zwait(i, carry):
            pltpu.make_async_copy(rows_v[0].at[pl.ds(0, piece)],
                                  acc_sh.at[pl.ds(0, piece)], semi[0]).wait()
            return carry

        lax.fori_loop(0, my_pieces, zwait, 0)
        if tail:
            @pl.when(s == NS - 1)
            def _():
                pltpu.sync_copy(rows_v[0].at[pl.ds(0, tail)],
                                acc_sh.at[pl.ds(n_full * piece, tail)])
        plsc.subcore_barrier()

        def ebody(t, carry):
            idxd = []
            for q in range(K):
                base = (w + (K * t + q) * NW) * CHUNK
                idxd.append((
                    pltpu.async_copy(src_hbm.at[pl.ds(base, CHUNK)], src_v[q], semi[q]),
                    pltpu.async_copy(dst_hbm.at[pl.ds(base, CHUNK)], dst_v[q], semi[q]),
                ))
            gd = []
            for q in range(K):
                idxd[q][0].wait()
                idxd[q][1].wait()
                gd.append(pltpu.async_copy(table_hbm.at[src_v[q]], rows_v[q], semg[q]))
            sd = []
            for q in range(K):
                gd[q].wait()
                sd.append(pltpu.async_copy(rows_v[q], acc_sh.at[dst_v[q]],
                                           sems[q], add=True))
            for q in range(K):
                sd[q].wait()
            return carry

        lax.fori_loop(0, n_iters, ebody, 0)

        for q in range(tail_iters):
            @pl.when(w + q * NW < rem)
            def _():
                base = (K * n_iters * NW + q * NW + w) * CHUNK
                pltpu.sync_copy(src_hbm.at[pl.ds(base, CHUNK)], src_v[0])
                pltpu.sync_copy(dst_hbm.at[pl.ds(base, CHUNK)], dst_v[0])
                pltpu.async_copy(table_hbm.at[src_v[0]], rows_v[0], semg[0]).wait()
                pltpu.sync_copy(rows_v[0], acc_sh.at[dst_v[0]], add=True)
        plsc.subcore_barrier()

        def obody(i, carry):
            r0 = (s + i * NS) * piece
            pltpu.async_copy(acc_sh.at[pl.ds(r0, piece)],
                             out_hbm.at[c].at[pl.ds(r0, piece)], sems[0])
            return carry

        lax.fori_loop(0, my_pieces, obody, 0)

        def owait(i, carry):
            pltpu.make_async_copy(acc_sh.at[pl.ds(0, piece)],
                                  out_hbm.at[c].at[pl.ds(0, piece)], sems[0]).wait()
            return carry

        lax.fori_loop(0, my_pieces, owait, 0)
        if tail:
            @pl.when(s == NS - 1)
            def _():
                r0 = n_full * piece
                pltpu.sync_copy(acc_sh.at[pl.ds(r0, tail)],
                                out_hbm.at[c].at[pl.ds(r0, tail)])

    return segsum(table, src, dst)


def _dense_tc(parts, w_mu, w_rho, eps_w, b_mu, b_rho, eps_b, relu):
    _, n, d = parts.shape
    blk = 1000
    assert n % blk == 0

    def body(p_ref, wmu_ref, wrho_ref, ew_ref, bmu_ref, brho_ref, eb_ref, o_ref):
        w = wmu_ref[...] + jnp.log1p(jnp.exp(wrho_ref[...])) * ew_ref[...]
        b = bmu_ref[...] + jnp.log1p(jnp.exp(brho_ref[...])) * eb_ref[...]
        a = p_ref[0] + p_ref[1]
        y = jnp.dot(a, w, preferred_element_type=jnp.float32) + b
        o_ref[...] = jnp.maximum(y, 0.0) if relu else y

    full = pl.BlockSpec((d, d), lambda i: (0, 0))
    vec = pl.BlockSpec((1, d), lambda i: (0, 0))
    return pl.pallas_call(
        body,
        grid=(n // blk,),
        in_specs=[
            pl.BlockSpec((2, blk, d), lambda i: (0, i, 0)),
            full, full, full, vec, vec, vec,
        ],
        out_specs=pl.BlockSpec((blk, d), lambda i: (i, 0)),
        out_shape=jax.ShapeDtypeStruct((n, d), jnp.float32),
    )(parts, w_mu, w_rho, eps_w,
      b_mu.reshape(1, d), b_rho.reshape(1, d), eps_b.reshape(1, d))


def kernel(x, edge_index, W1_mu, W1_rho, b1_mu, b1_rho, W2_mu, W2_rho, b2_mu, b2_rho):
    k = jax.random.key(42)
    k1, k2 = jax.random.split(k)
    kW1, kb1 = jax.random.split(k1)
    kW2, kb2 = jax.random.split(k2)
    eps_W1 = jax.random.normal(kW1, W1_mu.shape, W1_mu.dtype)
    eps_b1 = jax.random.normal(kb1, b1_mu.shape, b1_mu.dtype)
    eps_W2 = jax.random.normal(kW2, W2_mu.shape, W2_mu.dtype)
    eps_b2 = jax.random.normal(kb2, b2_mu.shape, b2_mu.dtype)

    src = edge_index[0]
    dst = edge_index[1]

    p1 = _segment_sum_sc(x, src, dst)
    h = _dense_tc(p1, W1_mu, W1_rho, eps_W1, b1_mu, b1_rho, eps_b1, relu=True)
    p2 = _segment_sum_sc(h, src, dst)
    return _dense_tc(p2, W2_mu, W2_rho, eps_W2, b2_mu, b2_rho, eps_b2, relu=False)
